# baseline (speedup 1.0000x reference)
import functools

import jax
import jax.numpy as jnp
from jax import lax
from jax.experimental import pallas as pl
from jax.experimental.pallas import tpu as pltpu
from jax.experimental.pallas import tpu_sc as plsc

M = 8
R = 8
V = 100000
D = 64
B = 1024
T = 50
LORA_ALPHA = 16.0
SCALING = LORA_ALPHA / R

N = B * T
NC, NS = 2, 16
NW = NC * NS
B_PER_W = N // NW
HALF = B_PER_W // 2
CH = 128


def _tc_prep(weight, lora_a_flat):
    vt = 8192
    grid = (pl.cdiv(V, vt),)

    def body(w_ref, a_ref, eye_ref, out_ref):
        stacked = jnp.concatenate([w_ref[...], a_ref[...]], axis=0)
        out_ref[...] = lax.dot_general(
            stacked, eye_ref[...], (((0,), (0,)), ((), ())),
            preferred_element_type=jnp.float32)

    return pl.pallas_call(
        body,
        grid=grid,
        in_specs=[
            pl.BlockSpec((D, vt), lambda i: (0, i)),
            pl.BlockSpec((M * R, vt), lambda i: (0, i)),
            pl.BlockSpec((2 * D, 2 * D), lambda i: (0, 0)),
        ],
        out_specs=pl.BlockSpec((vt, 2 * D), lambda i: (i, 0)),
        out_shape=jax.ShapeDtypeStruct((V, 2 * D), jnp.float32),
    )(weight, lora_a_flat, jnp.eye(2 * D, dtype=jnp.float32))


T1 = 15
T2 = T - T1


def _sc_gather(idx_flat, wc, off, nt, qch):
    mesh = plsc.VectorSubcoreMesh(core_axis_name="c", subcore_axis_name="s")
    per_w = nt * B // NW
    n_q = per_w // qch

    @functools.partial(
        pl.kernel,
        mesh=mesh,
        compiler_params=pltpu.CompilerParams(use_tc_tiling_on_sc=True),
        out_type=jax.ShapeDtypeStruct((nt * B, 2 * D), jnp.float32),
        scratch_types=[
            pltpu.VMEM((per_w,), jnp.int32),
            pltpu.VMEM((qch, 2 * D), jnp.float32),
            pltpu.VMEM((qch, 2 * D), jnp.float32),
            pltpu.SemaphoreType.DMA,
            pltpu.SemaphoreType.DMA,
        ],
    )
    def gather_kernel(idx_hbm, wc_hbm, g_hbm, idx_v, buf0, buf1, sem0, sem1):
        wid = lax.axis_index("s") * NC + lax.axis_index("c")
        base = wid * per_w
        pltpu.sync_copy(idx_hbm.at[pl.ds(off + base, per_w)], idx_v)
        bufs, sems = (buf0, buf1), (sem0, sem1)
        handles = [[] for _ in range(n_q)]
        for q in range(min(2, n_q)):
            for lo in range(0, qch, CH):
                sz = min(CH, qch - lo)
                handles[q].append(pltpu.async_copy(
                    wc_hbm.at[idx_v.at[pl.ds(q * qch + lo, sz)]],
                    bufs[q % 2].at[pl.ds(lo, sz)],
                    sems[q % 2],
                ))
        for q in range(n_q):
            for cp in handles[q]:
                cp.wait()
            pltpu.sync_copy(bufs[q % 2], g_hbm.at[pl.ds(base + q * qch, qch)])
            nq = q + 2
            if nq < n_q:
                for lo in range(0, qch, CH):
                    sz = min(CH, qch - lo)
                    handles[nq].append(pltpu.async_copy(
                        wc_hbm.at[idx_v.at[pl.ds(nq * qch + lo, sz)]],
                        bufs[nq % 2].at[pl.ds(lo, sz)],
                        sems[nq % 2],
                    ))

    return gather_kernel(idx_flat, wc)


TT = 5


def _tc_combine_first(g3, uft):
    def body(g_ref, u_ref, out_ref):
        for t in range(TT):
            g = g_ref[t]
            for m in range(M):
                out_ref[m, t] = lax.dot_general(
                    u_ref[m], g, (((1,), (1,)), ((), ())),
                    preferred_element_type=jnp.float32)

    return pl.pallas_call(
        body,
        grid=(T1 // TT,),
        in_specs=[
            pl.BlockSpec((TT, B, 2 * D), lambda i: (i, 0, 0)),
            pl.BlockSpec((M, D, 2 * D), lambda i: (0, 0, 0)),
        ],
        out_specs=pl.BlockSpec((M, TT, D, B), lambda i: (0, i, 0, 0)),
        out_shape=jax.ShapeDtypeStruct((M, T, D, B), jnp.float32),
    )(g3, uft)


def _tc_combine_second(prev, g3, uft):
    def body(prev_ref, g_ref, u_ref, out_ref):
        del prev_ref
        for t in range(TT):
            g = g_ref[t]
            for m in range(M):
                out_ref[m, t] = lax.dot_general(
                    u_ref[m], g, (((1,), (1,)), ((), ())),
                    preferred_element_type=jnp.float32)

    return pl.pallas_call(
        body,
        grid=(T2 // TT,),
        in_specs=[
            pl.BlockSpec(memory_space=pl.ANY),
            pl.BlockSpec((TT, B, 2 * D), lambda i: (i, 0, 0)),
            pl.BlockSpec((M, D, 2 * D), lambda i: (0, 0, 0)),
        ],
        out_specs=pl.BlockSpec((M, TT, D, B), lambda i: (0, i + T1 // TT, 0, 0)),
        out_shape=jax.ShapeDtypeStruct((M, T, D, B), jnp.float32),
        input_output_aliases={0: 0},
    )(prev, g3, uft)


def kernel(x, weight, lora_A, lora_B):
    idx_flat = jnp.swapaxes(x, 0, 1).reshape(N)
    wc = _tc_prep(jnp.transpose(weight), lora_A.reshape(M * R, V))
    g0 = _sc_gather(idx_flat, wc, 0, T1, 240)
    g1 = _sc_gather(idx_flat, wc, T1 * B, T2, 280)

    p = SCALING * jnp.transpose(lora_B, (0, 2, 1))
    p_tiled = jnp.tile(p, (1, M, 1))
    sel = (jnp.arange(M * R)[None, :, None] // R
           == jnp.arange(M)[:, None, None])
    p_big = jnp.where(sel, p_tiled, 0.0)
    eye = jnp.broadcast_to(jnp.eye(D, dtype=jnp.float32), (M, D, D))
    u = jnp.concatenate([eye, p_big], axis=1)
    uft = jnp.transpose(u, (0, 2, 1))

    out_a = _tc_combine_first(g0.reshape(T1, B, 2 * D), uft)
    out_phys = _tc_combine_second(out_a, g1.reshape(T2, B, 2 * D), uft)
    return jnp.transpose(out_phys, (0, 3, 1, 2))


# speedup vs baseline: 1.0207x; 1.0207x over previous
import functools

import jax
import jax.numpy as jnp
from jax import lax
from jax.experimental import pallas as pl
from jax.experimental.pallas import tpu as pltpu
from jax.experimental.pallas import tpu_sc as plsc

M = 8
R = 8
V = 100000
D = 64
B = 1024
T = 50
LORA_ALPHA = 16.0
SCALING = LORA_ALPHA / R

N = B * T
NC, NS = 2, 16
NW = NC * NS
B_PER_W = N // NW
HALF = B_PER_W // 2
CH = 128


def _tc_prep(weight, lora_a_flat):
    vt = 8192
    grid = (pl.cdiv(V, vt),)

    def body(w_ref, a_ref, eye_ref, out_ref):
        stacked = jnp.concatenate([w_ref[...], a_ref[...]], axis=0)
        out_ref[...] = lax.dot_general(
            stacked, eye_ref[...], (((0,), (0,)), ((), ())),
            preferred_element_type=jnp.float32)

    return pl.pallas_call(
        body,
        grid=grid,
        in_specs=[
            pl.BlockSpec((D, vt), lambda i: (0, i)),
            pl.BlockSpec((M * R, vt), lambda i: (0, i)),
            pl.BlockSpec((2 * D, 2 * D), lambda i: (0, 0)),
        ],
        out_specs=pl.BlockSpec((vt, 2 * D), lambda i: (i, 0)),
        out_shape=jax.ShapeDtypeStruct((V, 2 * D), jnp.float32),
    )(weight, lora_a_flat, jnp.eye(2 * D, dtype=jnp.float32))


T1 = T
T2 = T - T1


def _sc_gather(idx_flat, wc, off, nt, qch):
    mesh = plsc.VectorSubcoreMesh(core_axis_name="c", subcore_axis_name="s")
    per_w = nt * B // NW
    n_q = per_w // qch

    @functools.partial(
        pl.kernel,
        mesh=mesh,
        compiler_params=pltpu.CompilerParams(use_tc_tiling_on_sc=True),
        out_type=jax.ShapeDtypeStruct((nt * B, 2 * D), jnp.float32),
        scratch_types=[
            pltpu.VMEM((per_w,), jnp.int32),
            pltpu.VMEM((qch, 2 * D), jnp.float32),
            pltpu.VMEM((qch, 2 * D), jnp.float32),
            pltpu.SemaphoreType.DMA,
            pltpu.SemaphoreType.DMA,
        ],
    )
    def gather_kernel(idx_hbm, wc_hbm, g_hbm, idx_v, buf0, buf1, sem0, sem1):
        wid = lax.axis_index("s") * NC + lax.axis_index("c")
        base = wid * per_w
        pltpu.sync_copy(idx_hbm.at[pl.ds(off + base, per_w)], idx_v)
        bufs, sems = (buf0, buf1), (sem0, sem1)
        handles = [[] for _ in range(n_q)]
        for q in range(min(2, n_q)):
            for lo in range(0, qch, CH):
                sz = min(CH, qch - lo)
                handles[q].append(pltpu.async_copy(
                    wc_hbm.at[idx_v.at[pl.ds(q * qch + lo, sz)]],
                    bufs[q % 2].at[pl.ds(lo, sz)],
                    sems[q % 2],
                ))
        for q in range(n_q):
            for cp in handles[q]:
                cp.wait()
            pltpu.sync_copy(bufs[q % 2], g_hbm.at[pl.ds(base + q * qch, qch)])
            nq = q + 2
            if nq < n_q:
                for lo in range(0, qch, CH):
                    sz = min(CH, qch - lo)
                    handles[nq].append(pltpu.async_copy(
                        wc_hbm.at[idx_v.at[pl.ds(nq * qch + lo, sz)]],
                        bufs[nq % 2].at[pl.ds(lo, sz)],
                        sems[nq % 2],
                    ))

    return gather_kernel(idx_flat, wc)


TT = 5


def _tc_combine_first(g3, uft):
    def body(g_ref, u_ref, out_ref):
        for t in range(TT):
            g = g_ref[t]
            for m in range(M):
                out_ref[m, t] = lax.dot_general(
                    u_ref[m], g, (((1,), (1,)), ((), ())),
                    preferred_element_type=jnp.float32)

    return pl.pallas_call(
        body,
        grid=(T1 // TT,),
        in_specs=[
            pl.BlockSpec((TT, B, 2 * D), lambda i: (i, 0, 0)),
            pl.BlockSpec((M, D, 2 * D), lambda i: (0, 0, 0)),
        ],
        out_specs=pl.BlockSpec((M, TT, D, B), lambda i: (0, i, 0, 0)),
        out_shape=jax.ShapeDtypeStruct((M, T, D, B), jnp.float32),
    )(g3, uft)


def _tc_combine_second(prev, g3, uft):
    def body(prev_ref, g_ref, u_ref, out_ref):
        del prev_ref
        for t in range(TT):
            g = g_ref[t]
            for m in range(M):
                out_ref[m, t] = lax.dot_general(
                    u_ref[m], g, (((1,), (1,)), ((), ())),
                    preferred_element_type=jnp.float32)

    return pl.pallas_call(
        body,
        grid=(T2 // TT,),
        in_specs=[
            pl.BlockSpec(memory_space=pl.ANY),
            pl.BlockSpec((TT, B, 2 * D), lambda i: (i, 0, 0)),
            pl.BlockSpec((M, D, 2 * D), lambda i: (0, 0, 0)),
        ],
        out_specs=pl.BlockSpec((M, TT, D, B), lambda i: (0, i + T1 // TT, 0, 0)),
        out_shape=jax.ShapeDtypeStruct((M, T, D, B), jnp.float32),
        input_output_aliases={0: 0},
    )(prev, g3, uft)


def kernel(x, weight, lora_A, lora_B):
    idx_flat = jnp.swapaxes(x, 0, 1).reshape(N)
    wc = _tc_prep(jnp.transpose(weight), lora_A.reshape(M * R, V))
    g0 = _sc_gather(idx_flat, wc, 0, T1, 400)
    g1 = _sc_gather(idx_flat, wc, T1 * B, T2, 280) if T2 else None

    p = SCALING * jnp.transpose(lora_B, (0, 2, 1))
    p_tiled = jnp.tile(p, (1, M, 1))
    sel = (jnp.arange(M * R)[None, :, None] // R
           == jnp.arange(M)[:, None, None])
    p_big = jnp.where(sel, p_tiled, 0.0)
    eye = jnp.broadcast_to(jnp.eye(D, dtype=jnp.float32), (M, D, D))
    u = jnp.concatenate([eye, p_big], axis=1)
    uft = jnp.transpose(u, (0, 2, 1))

    out_a = _tc_combine_first(g0.reshape(T1, B, 2 * D), uft)
    out_phys = (_tc_combine_second(out_a, g1.reshape(T2, B, 2 * D), uft)
                if T2 else out_a)
    return jnp.transpose(out_phys, (0, 3, 1, 2))
